# Initial kernel scaffold; baseline (speedup 1.0000x reference)
import jax
import jax.numpy as jnp
from jax.experimental import pallas as pl


def kernel(x, edge_index, W1, att_src1, att_dst1, b1, W2, att_src2, att_dst2, b2):
    raise NotImplementedError("write your pallas kernel here")


# speedup vs baseline: 55.8625x; 55.8625x over previous
import functools

import jax
import jax.numpy as jnp
from jax import lax
from jax.experimental import pallas as pl
from jax.experimental.pallas import tpu as pltpu
from jax.experimental.pallas import tpu_sc as plsc

F32 = jnp.float32
I32 = jnp.int32
BIG = 1e30
NC = 2
NS = 16
NW = NC * NS
K = 128


def _lrelu(v):
    return jnp.where(v > 0, v, 0.2 * v)


def _bcast_lane(vec, lane):
    idx = jnp.full((16, 1), lane, I32)
    dnums = lax.GatherDimensionNumbers(
        offset_dims=(), collapsed_slice_dims=(0,), start_index_map=(0,))
    return lax.gather(vec, idx, dnums, (1,),
                      mode=lax.GatherScatterMode.PROMISE_IN_BOUNDS)


def _tc_prep1(x, W1, Ms, Md, Rexp, npad_edges):
    n = x.shape[0]

    def body(x_ref, w_ref, ms_ref, md_ref, rexp_ref, stab_ref, dtab_ref,
             corr_ref):
        h = jnp.dot(x_ref[...], w_ref[...], preferred_element_type=F32)
        a_s = jnp.dot(h, ms_ref[...], preferred_element_type=F32)
        a_d = jnp.dot(h, md_ref[...], preferred_element_type=F32)
        smax = jnp.max(a_s, axis=0, keepdims=True)
        c = _lrelu(a_d + smax)
        z8 = jnp.zeros((n, 8), F32)
        stab_ref[...] = jnp.concatenate([h, a_s, z8], axis=1)
        dtab_ref[...] = jnp.concatenate(
            [a_d, z8, c, jnp.full((n, 8), BIG, F32)], axis=1)
        pex0 = float(npad_edges) * jnp.exp(
            _lrelu(a_s[0:1, :] + a_d[0:1, :]) - c[0:1, :])
        pex0_exp = jnp.dot(pex0, rexp_ref[...],
                           preferred_element_type=F32)
        corr_ref[...] = jnp.concatenate(
            [pex0_exp * h[0:1, :], pex0, jnp.zeros((1, 8), F32)], axis=1)

    return pl.pallas_call(
        body,
        out_shape=(jax.ShapeDtypeStruct((n, 144), F32),
                   jax.ShapeDtypeStruct((n, 32), F32),
                   jax.ShapeDtypeStruct((1, 144), F32)))(x, W1, Ms, Md, Rexp)


def _tc_prep2(accden, corr1, b1, W2, As2t, Ad2t, R16, n, npad_edges):
    def body(ad_ref, corr_ref, b1_ref, w2_ref, as2_ref, ad2_ref, r16_ref,
             stab_ref, dtab_ref, corr2_ref):
        A = ad_ref[0] + ad_ref[1]
        A = A - jnp.concatenate(
            [corr_ref[...], jnp.zeros((n - 1, 144), F32)], axis=0)
        acc = A[:, 0:128]
        den = A[:, 128:144]
        den_exp = jnp.dot(den, r16_ref[...], preferred_element_type=F32)
        h1 = acc / (den_exp + 1e-16) + b1_ref[...]
        h1 = jnp.where(h1 > 0, h1, jnp.exp(h1) - 1.0)
        h2 = jnp.dot(h1, w2_ref[...], preferred_element_type=F32)
        a2s = jnp.dot(h2, as2_ref[...], preferred_element_type=F32)
        a2d = jnp.dot(h2, ad2_ref[...], preferred_element_type=F32)
        smax = jnp.max(a2s)
        c2 = _lrelu(a2d + smax)
        stab_ref[...] = jnp.concatenate(
            [h2, a2s, jnp.zeros((n, 7), F32)], axis=1)
        dtab_ref[...] = jnp.concatenate(
            [jnp.zeros((n, 8), F32), a2d, jnp.zeros((n, 7), F32),
             jnp.full((n, 8), BIG, F32), c2, jnp.full((n, 7), BIG, F32)],
            axis=1)
        pe0 = float(npad_edges) * jnp.exp(
            _lrelu(a2s[0:1, :] + a2d[0:1, :]) - c2[0:1, :])
        corr2_ref[...] = jnp.concatenate(
            [pe0 * h2[0:1, :], jnp.broadcast_to(pe0, (1, 8))], axis=1)

    return pl.pallas_call(
        body,
        out_shape=(jax.ShapeDtypeStruct((n, 48), F32),
                   jax.ShapeDtypeStruct((n, 32), F32),
                   jax.ShapeDtypeStruct((1, 48), F32)))(
            accden, corr1, b1, W2, As2t, Ad2t, R16)


def _tc_final(accden, corr2, b2, Rden, n):
    def body(ad_ref, corr_ref, b2_ref, rd_ref, out_ref):
        A = ad_ref[0] + ad_ref[1]
        A = A - jnp.concatenate(
            [corr_ref[...], jnp.zeros((n - 1, 48), F32)], axis=0)
        acc = A[:, 0:40]
        den = jnp.dot(A[:, 32:48], rd_ref[...], preferred_element_type=F32)
        out_ref[...] = acc / (den + 1e-16) + b2_ref[...]

    return pl.pallas_call(
        body, out_shape=jax.ShapeDtypeStruct((n, 40), F32))(
            accden, corr2, b2, Rden)


def _sc_edge1(stab, dtab, pk, n_pad, chunks):
    rows_pt = n_pad // NS
    mesh = plsc.VectorSubcoreMesh(core_axis_name="c", subcore_axis_name="s")

    @functools.partial(
        pl.kernel,
        out_type=jax.ShapeDtypeStruct((NC, n_pad, 144), F32),
        mesh=mesh,
        compiler_params=pltpu.CompilerParams(use_tc_tiling_on_sc=False),
        scratch_types=[
            pltpu.VMEM((K,), I32),
            pltpu.VMEM((K,), I32),
            pltpu.VMEM((K,), I32),
            pltpu.VMEM((K, 144), F32),
            pltpu.VMEM((K, 32), F32),
            pltpu.VMEM_SHARED((n_pad, 144), F32),
            pltpu.SemaphoreType.DMA,
            pltpu.SemaphoreType.DMA,
        ])
    def kfn(stab_hbm, dtab_hbm, pk_hbm, out_hbm,
            pkbuf, sidx, didx, srows, drows, acc, sem1, sem2):
        cid = lax.axis_index("c")
        sid = lax.axis_index("s")
        wid = cid * NS + sid
        zero = jnp.zeros((16,), F32)

        def zbody(r, carry):
            for cc in range(9):
                srows[r, pl.ds(cc * 16, 16)] = zero
            return carry
        lax.fori_loop(0, K, zbody, 0)
        base0 = sid * rows_pt
        nfull, rem = rows_pt // K, rows_pt % K
        for i in range(nfull):
            pltpu.sync_copy(srows, acc.at[pl.ds(base0 + i * K, K)])
        if rem:
            pltpu.sync_copy(srows.at[pl.ds(0, rem)],
                            acc.at[pl.ds(base0 + nfull * K, rem)])
        plsc.subcore_barrier()

        def chunk(g, carry):
            base = wid * (chunks * K) + g * K
            pltpu.sync_copy(pk_hbm.at[pl.ds(base, K)], pkbuf)
            for l in range(K // 16):
                v = pkbuf[pl.ds(l * 16, 16)]
                sidx[pl.ds(l * 16, 16)] = lax.shift_right_logical(v, 14)
                didx[pl.ds(l * 16, 16)] = lax.bitwise_and(v, 16383)
            cp1 = pltpu.async_copy(stab_hbm.at[sidx], srows, sem1)
            cp2 = pltpu.async_copy(dtab_hbm.at[didx], drows, sem2)
            cp1.wait()
            cp2.wait()

            def ebody(j, ecarry):
                av = srows[j, pl.ds(128, 16)]
                d0 = drows[j, pl.ds(0, 16)]
                d1 = drows[j, pl.ds(16, 16)]
                s = av + d0
                ex = jnp.exp(jnp.where(s > 0, s, 0.2 * s) - d1)
                srows[j, pl.ds(128, 16)] = ex
                for hh in range(8):
                    exb = _bcast_lane(ex, hh)
                    srows[j, pl.ds(hh * 16, 16)] = (
                        srows[j, pl.ds(hh * 16, 16)] * exb)
                return ecarry
            lax.fori_loop(0, K, ebody, 0)
            pltpu.sync_copy(srows, acc.at[didx], add=True)
            return carry
        lax.fori_loop(0, chunks, chunk, 0)

        plsc.subcore_barrier()
        pltpu.sync_copy(acc.at[pl.ds(base0, rows_pt)],
                        out_hbm.at[cid, pl.ds(base0, rows_pt)])

    return kfn(stab, dtab, pk)


def _sc_edge2(stab, dtab, pk, n_pad, chunks):
    rows_pt = n_pad // NS
    mesh = plsc.VectorSubcoreMesh(core_axis_name="c", subcore_axis_name="s")

    @functools.partial(
        pl.kernel,
        out_type=jax.ShapeDtypeStruct((NC, n_pad, 48), F32),
        mesh=mesh,
        compiler_params=pltpu.CompilerParams(use_tc_tiling_on_sc=False),
        scratch_types=[
            pltpu.VMEM((K,), I32),
            pltpu.VMEM((K,), I32),
            pltpu.VMEM((K,), I32),
            pltpu.VMEM((K, 48), F32),
            pltpu.VMEM((K, 32), F32),
            pltpu.VMEM_SHARED((n_pad, 48), F32),
            pltpu.SemaphoreType.DMA,
            pltpu.SemaphoreType.DMA,
        ])
    def kfn(stab_hbm, dtab_hbm, pk_hbm, out_hbm,
            pkbuf, sidx, didx, srows, drows, acc, sem1, sem2):
        cid = lax.axis_index("c")
        sid = lax.axis_index("s")
        wid = cid * NS + sid
        zero = jnp.zeros((16,), F32)

        def zbody(r, carry):
            for cc in range(3):
                srows[r, pl.ds(cc * 16, 16)] = zero
            return carry
        lax.fori_loop(0, K, zbody, 0)
        base0 = sid * rows_pt
        nfull, rem = rows_pt // K, rows_pt % K
        for i in range(nfull):
            pltpu.sync_copy(srows, acc.at[pl.ds(base0 + i * K, K)])
        if rem:
            pltpu.sync_copy(srows.at[pl.ds(0, rem)],
                            acc.at[pl.ds(base0 + nfull * K, rem)])
        plsc.subcore_barrier()

        def chunk(g, carry):
            base = wid * (chunks * K) + g * K
            pltpu.sync_copy(pk_hbm.at[pl.ds(base, K)], pkbuf)
            for l in range(K // 16):
                v = pkbuf[pl.ds(l * 16, 16)]
                sidx[pl.ds(l * 16, 16)] = lax.shift_right_logical(v, 14)
                didx[pl.ds(l * 16, 16)] = lax.bitwise_and(v, 16383)
            cp1 = pltpu.async_copy(stab_hbm.at[sidx], srows, sem1)
            cp2 = pltpu.async_copy(dtab_hbm.at[didx], drows, sem2)
            cp1.wait()
            cp2.wait()

            lanes = lax.iota(I32, 16)

            def ebody(j, ecarry):
                sv2 = srows[j, pl.ds(32, 16)]
                d0 = drows[j, pl.ds(0, 16)]
                d1 = drows[j, pl.ds(16, 16)]
                t = sv2 + d0
                e0 = jnp.exp(jnp.where(t > 0, t, 0.2 * t) - d1)
                exv = _bcast_lane(e0, 8)
                srows[j, pl.ds(0, 16)] = srows[j, pl.ds(0, 16)] * exv
                srows[j, pl.ds(16, 16)] = srows[j, pl.ds(16, 16)] * exv
                srows[j, pl.ds(32, 16)] = jnp.where(lanes < 8, sv2 * exv, exv)
                return ecarry
            lax.fori_loop(0, K, ebody, 0)
            pltpu.sync_copy(srows, acc.at[didx], add=True)
            return carry
        lax.fori_loop(0, chunks, chunk, 0)

        plsc.subcore_barrier()
        pltpu.sync_copy(acc.at[pl.ds(base0, rows_pt)],
                        out_hbm.at[cid, pl.ds(base0, rows_pt)])

    return kfn(stab, dtab, pk)


def kernel(x, edge_index, W1, att_src1, att_dst1, b1,
           W2, att_src2, att_dst2, b2):
    n = x.shape[0]
    e = edge_index.shape[1]
    heads, hid = att_src1.shape

    et = e + n
    ep = ((et + NW * K - 1) // (NW * K)) * (NW * K)
    chunks = ep // (NW * K)
    npad_edges = ep - et

    loops = jnp.arange(n, dtype=I32)
    padi = jnp.zeros((npad_edges,), I32)
    src = jnp.concatenate([edge_index[0].astype(I32), loops, padi])
    dst = jnp.concatenate([edge_index[1].astype(I32), loops, padi])
    pk = jnp.bitwise_or(jnp.left_shift(src, 14), dst)

    eye8 = jnp.eye(heads, dtype=F32)
    Ms1 = (att_src1.astype(F32)[:, :, None] * eye8[:, None, :]).reshape(
        heads * hid, heads)
    Md1 = (att_dst1.astype(F32)[:, :, None] * eye8[:, None, :]).reshape(
        heads * hid, heads)
    Rexp = jnp.repeat(eye8, hid, axis=1)
    R16 = jnp.concatenate(
        [Rexp, jnp.zeros((8, heads * hid), F32)], axis=0)
    Rden = jnp.concatenate(
        [jnp.zeros((8, 40), F32), jnp.full((8, 40), 1.0 / 8.0, F32)], axis=0)

    stab1, dtab1, corr1 = _tc_prep1(x, W1, Ms1, Md1, Rexp, npad_edges)
    accden1 = _sc_edge1(stab1, dtab1, pk, n, chunks)
    stab2, dtab2, corr2 = _tc_prep2(accden1, corr1, b1, W2,
                                    att_src2.T.astype(F32),
                                    att_dst2.T.astype(F32), R16, n,
                                    npad_edges)
    accden2 = _sc_edge2(stab2, dtab2, pk, n, chunks)
    return _tc_final(accden2, corr2, b2, Rden, n)


# speedup vs baseline: 79.4654x; 1.4225x over previous
import functools

import jax
import jax.numpy as jnp
from jax import lax
from jax.experimental import pallas as pl
from jax.experimental.pallas import tpu as pltpu
from jax.experimental.pallas import tpu_sc as plsc

F32 = jnp.float32
I32 = jnp.int32
BIG = 1e30
NC = 2
NS = 16
NW = NC * NS
K = 128


def _lrelu(v):
    return jnp.where(v > 0, v, 0.2 * v)


def _bcast_lane(vec, lane):
    idx = jnp.full((16, 1), lane, I32)
    dnums = lax.GatherDimensionNumbers(
        offset_dims=(), collapsed_slice_dims=(0,), start_index_map=(0,))
    return lax.gather(vec, idx, dnums, (1,),
                      mode=lax.GatherScatterMode.PROMISE_IN_BOUNDS)


def _tc_prep1(x, W1, Ms, Md, Rexp, npad_edges):
    n = x.shape[0]

    def body(x_ref, w_ref, ms_ref, md_ref, rexp_ref, stab_ref, dtab_ref,
             corr_ref):
        h = jnp.dot(x_ref[...], w_ref[...], preferred_element_type=F32)
        a_s = jnp.dot(h, ms_ref[...], preferred_element_type=F32)
        a_d = jnp.dot(h, md_ref[...], preferred_element_type=F32)
        smax = jnp.max(a_s, axis=0, keepdims=True)
        c = _lrelu(a_d + smax)
        z8 = jnp.zeros((n, 8), F32)
        stab_ref[...] = jnp.concatenate([h, a_s, z8], axis=1)
        dtab_ref[...] = jnp.concatenate(
            [a_d, z8, c, jnp.full((n, 8), BIG, F32)], axis=1)
        pex0 = float(npad_edges) * jnp.exp(
            _lrelu(a_s[0:1, :] + a_d[0:1, :]) - c[0:1, :])
        pex0_exp = jnp.dot(pex0, rexp_ref[...],
                           preferred_element_type=F32)
        corr_ref[...] = jnp.concatenate(
            [pex0_exp * h[0:1, :], pex0, jnp.zeros((1, 8), F32)], axis=1)

    return pl.pallas_call(
        body,
        out_shape=(jax.ShapeDtypeStruct((n, 144), F32),
                   jax.ShapeDtypeStruct((n, 32), F32),
                   jax.ShapeDtypeStruct((1, 144), F32)))(x, W1, Ms, Md, Rexp)


def _tc_prep2(accden, corr1, b1, W2, As2t, Ad2t, R16, n, npad_edges):
    def body(ad_ref, corr_ref, b1_ref, w2_ref, as2_ref, ad2_ref, r16_ref,
             stab_ref, dtab_ref, corr2_ref):
        A = ad_ref[0] + ad_ref[1]
        A = A - jnp.concatenate(
            [corr_ref[...], jnp.zeros((n - 1, 144), F32)], axis=0)
        acc = A[:, 0:128]
        den = A[:, 128:144]
        den_exp = jnp.dot(den, r16_ref[...], preferred_element_type=F32)
        h1 = acc / (den_exp + 1e-16) + b1_ref[...]
        h1 = jnp.where(h1 > 0, h1, jnp.exp(h1) - 1.0)
        h2 = jnp.dot(h1, w2_ref[...], preferred_element_type=F32)
        a2s = jnp.dot(h2, as2_ref[...], preferred_element_type=F32)
        a2d = jnp.dot(h2, ad2_ref[...], preferred_element_type=F32)
        smax = jnp.max(a2s)
        c2 = _lrelu(a2d + smax)
        stab_ref[...] = jnp.concatenate(
            [h2, a2s, jnp.zeros((n, 7), F32)], axis=1)
        dtab_ref[...] = jnp.concatenate(
            [jnp.zeros((n, 8), F32), a2d, jnp.zeros((n, 7), F32),
             jnp.full((n, 8), BIG, F32), c2, jnp.full((n, 7), BIG, F32)],
            axis=1)
        pe0 = float(npad_edges) * jnp.exp(
            _lrelu(a2s[0:1, :] + a2d[0:1, :]) - c2[0:1, :])
        corr2_ref[...] = jnp.concatenate(
            [pe0 * h2[0:1, :], jnp.broadcast_to(pe0, (1, 8))], axis=1)

    return pl.pallas_call(
        body,
        out_shape=(jax.ShapeDtypeStruct((n, 48), F32),
                   jax.ShapeDtypeStruct((n, 32), F32),
                   jax.ShapeDtypeStruct((1, 48), F32)))(
            accden, corr1, b1, W2, As2t, Ad2t, R16)


def _tc_final(accden, corr2, b2, Rden, n):
    def body(ad_ref, corr_ref, b2_ref, rd_ref, out_ref):
        A = ad_ref[0] + ad_ref[1]
        A = A - jnp.concatenate(
            [corr_ref[...], jnp.zeros((n - 1, 48), F32)], axis=0)
        acc = A[:, 0:40]
        den = jnp.dot(A[:, 32:48], rd_ref[...], preferred_element_type=F32)
        out_ref[...] = acc / (den + 1e-16) + b2_ref[...]

    return pl.pallas_call(
        body, out_shape=jax.ShapeDtypeStruct((n, 40), F32))(
            accden, corr2, b2, Rden)


def _sc_edge1(stab, dtab, pk, n_pad, chunks):
    rows_pt = n_pad // NS
    mesh = plsc.VectorSubcoreMesh(core_axis_name="c", subcore_axis_name="s")

    @functools.partial(
        pl.kernel,
        out_type=jax.ShapeDtypeStruct((NC, n_pad, 144), F32),
        mesh=mesh,
        compiler_params=pltpu.CompilerParams(use_tc_tiling_on_sc=False),
        scratch_types=[
            pltpu.VMEM((K,), I32),
            pltpu.VMEM((K,), I32),
            pltpu.VMEM((K,), I32),
            pltpu.VMEM((K, 144), F32),
            pltpu.VMEM((K, 32), F32),
            pltpu.VMEM_SHARED((n_pad, 144), F32),
            pltpu.SemaphoreType.DMA,
            pltpu.SemaphoreType.DMA,
        ])
    def kfn(stab_hbm, dtab_hbm, pk_hbm, out_hbm,
            pkbuf, sidx, didx, srows, drows, acc, sem1, sem2):
        cid = lax.axis_index("c")
        sid = lax.axis_index("s")
        wid = cid * NS + sid
        zero = jnp.zeros((16,), F32)

        @plsc.parallel_loop(0, K, unroll=8)
        def zbody(r):
            for cc in range(9):
                srows[r, pl.ds(cc * 16, 16)] = zero
        base0 = sid * rows_pt
        nfull, rem = rows_pt // K, rows_pt % K
        for i in range(nfull):
            pltpu.sync_copy(srows, acc.at[pl.ds(base0 + i * K, K)])
        if rem:
            pltpu.sync_copy(srows.at[pl.ds(0, rem)],
                            acc.at[pl.ds(base0 + nfull * K, rem)])
        plsc.subcore_barrier()

        def chunk(g, carry):
            base = wid * (chunks * K) + g * K
            pltpu.sync_copy(pk_hbm.at[pl.ds(base, K)], pkbuf)
            for l in range(K // 16):
                v = pkbuf[pl.ds(l * 16, 16)]
                sidx[pl.ds(l * 16, 16)] = lax.shift_right_logical(v, 14)
                didx[pl.ds(l * 16, 16)] = lax.bitwise_and(v, 16383)
            cp1 = pltpu.async_copy(stab_hbm.at[sidx], srows, sem1)
            cp2 = pltpu.async_copy(dtab_hbm.at[didx], drows, sem2)
            cp1.wait()
            cp2.wait()

            @plsc.parallel_loop(0, K, unroll=4)
            def ebody(j):
                av = srows[j, pl.ds(128, 16)]
                d0 = drows[j, pl.ds(0, 16)]
                d1 = drows[j, pl.ds(16, 16)]
                s = av + d0
                ex = jnp.exp(jnp.where(s > 0, s, 0.2 * s) - d1)
                srows[j, pl.ds(128, 16)] = ex
                for hh in range(8):
                    exb = _bcast_lane(ex, hh)
                    srows[j, pl.ds(hh * 16, 16)] = (
                        srows[j, pl.ds(hh * 16, 16)] * exb)
            pltpu.sync_copy(srows, acc.at[didx], add=True)
            return carry
        lax.fori_loop(0, chunks, chunk, 0)

        plsc.subcore_barrier()
        pltpu.sync_copy(acc.at[pl.ds(base0, rows_pt)],
                        out_hbm.at[cid, pl.ds(base0, rows_pt)])

    return kfn(stab, dtab, pk)


def _sc_edge2(stab, dtab, pk, n_pad, chunks):
    rows_pt = n_pad // NS
    mesh = plsc.VectorSubcoreMesh(core_axis_name="c", subcore_axis_name="s")

    @functools.partial(
        pl.kernel,
        out_type=jax.ShapeDtypeStruct((NC, n_pad, 48), F32),
        mesh=mesh,
        compiler_params=pltpu.CompilerParams(use_tc_tiling_on_sc=False),
        scratch_types=[
            pltpu.VMEM((K,), I32),
            pltpu.VMEM((K,), I32),
            pltpu.VMEM((K,), I32),
            pltpu.VMEM((K, 48), F32),
            pltpu.VMEM((K, 32), F32),
            pltpu.VMEM_SHARED((n_pad, 48), F32),
            pltpu.SemaphoreType.DMA,
            pltpu.SemaphoreType.DMA,
        ])
    def kfn(stab_hbm, dtab_hbm, pk_hbm, out_hbm,
            pkbuf, sidx, didx, srows, drows, acc, sem1, sem2):
        cid = lax.axis_index("c")
        sid = lax.axis_index("s")
        wid = cid * NS + sid
        zero = jnp.zeros((16,), F32)

        @plsc.parallel_loop(0, K, unroll=8)
        def zbody(r):
            for cc in range(3):
                srows[r, pl.ds(cc * 16, 16)] = zero
        base0 = sid * rows_pt
        nfull, rem = rows_pt // K, rows_pt % K
        for i in range(nfull):
            pltpu.sync_copy(srows, acc.at[pl.ds(base0 + i * K, K)])
        if rem:
            pltpu.sync_copy(srows.at[pl.ds(0, rem)],
                            acc.at[pl.ds(base0 + nfull * K, rem)])
        plsc.subcore_barrier()

        def chunk(g, carry):
            base = wid * (chunks * K) + g * K
            pltpu.sync_copy(pk_hbm.at[pl.ds(base, K)], pkbuf)
            for l in range(K // 16):
                v = pkbuf[pl.ds(l * 16, 16)]
                sidx[pl.ds(l * 16, 16)] = lax.shift_right_logical(v, 14)
                didx[pl.ds(l * 16, 16)] = lax.bitwise_and(v, 16383)
            cp1 = pltpu.async_copy(stab_hbm.at[sidx], srows, sem1)
            cp2 = pltpu.async_copy(dtab_hbm.at[didx], drows, sem2)
            cp1.wait()
            cp2.wait()

            lanes = lax.iota(I32, 16)

            @plsc.parallel_loop(0, K, unroll=4)
            def ebody(j):
                sv2 = srows[j, pl.ds(32, 16)]
                d0 = drows[j, pl.ds(0, 16)]
                d1 = drows[j, pl.ds(16, 16)]
                t = sv2 + d0
                e0 = jnp.exp(jnp.where(t > 0, t, 0.2 * t) - d1)
                exv = _bcast_lane(e0, 8)
                srows[j, pl.ds(0, 16)] = srows[j, pl.ds(0, 16)] * exv
                srows[j, pl.ds(16, 16)] = srows[j, pl.ds(16, 16)] * exv
                srows[j, pl.ds(32, 16)] = jnp.where(lanes < 8, sv2 * exv, exv)
            pltpu.sync_copy(srows, acc.at[didx], add=True)
            return carry
        lax.fori_loop(0, chunks, chunk, 0)

        plsc.subcore_barrier()
        pltpu.sync_copy(acc.at[pl.ds(base0, rows_pt)],
                        out_hbm.at[cid, pl.ds(base0, rows_pt)])

    return kfn(stab, dtab, pk)


def kernel(x, edge_index, W1, att_src1, att_dst1, b1,
           W2, att_src2, att_dst2, b2):
    n = x.shape[0]
    e = edge_index.shape[1]
    heads, hid = att_src1.shape

    et = e + n
    ep = ((et + NW * K - 1) // (NW * K)) * (NW * K)
    chunks = ep // (NW * K)
    npad_edges = ep - et

    loops = jnp.arange(n, dtype=I32)
    padi = jnp.zeros((npad_edges,), I32)
    src = jnp.concatenate([edge_index[0].astype(I32), loops, padi])
    dst = jnp.concatenate([edge_index[1].astype(I32), loops, padi])
    pk = jnp.bitwise_or(jnp.left_shift(src, 14), dst)

    eye8 = jnp.eye(heads, dtype=F32)
    Ms1 = (att_src1.astype(F32)[:, :, None] * eye8[:, None, :]).reshape(
        heads * hid, heads)
    Md1 = (att_dst1.astype(F32)[:, :, None] * eye8[:, None, :]).reshape(
        heads * hid, heads)
    Rexp = jnp.repeat(eye8, hid, axis=1)
    R16 = jnp.concatenate(
        [Rexp, jnp.zeros((8, heads * hid), F32)], axis=0)
    Rden = jnp.concatenate(
        [jnp.zeros((8, 40), F32), jnp.full((8, 40), 1.0 / 8.0, F32)], axis=0)

    stab1, dtab1, corr1 = _tc_prep1(x, W1, Ms1, Md1, Rexp, npad_edges)
    accden1 = _sc_edge1(stab1, dtab1, pk, n, chunks)
    stab2, dtab2, corr2 = _tc_prep2(accden1, corr1, b1, W2,
                                    att_src2.T.astype(F32),
                                    att_dst2.T.astype(F32), R16, n,
                                    npad_edges)
    accden2 = _sc_edge2(stab2, dtab2, pk, n, chunks)
    return _tc_final(accden2, corr2, b2, Rden, n)


# speedup vs baseline: 113.3952x; 1.4270x over previous
import functools

import jax
import jax.numpy as jnp
from jax import lax
from jax.experimental import pallas as pl
from jax.experimental.pallas import tpu as pltpu
from jax.experimental.pallas import tpu_sc as plsc

F32 = jnp.float32
I32 = jnp.int32
BIG = 1e30
NC = 2
NS = 16
NW = NC * NS
K = 64


def _lrelu(v):
    return jnp.where(v > 0, v, 0.2 * v)


def _bcast_lane(vec, lane):
    idx = jnp.full((16, 1), lane, I32)
    dnums = lax.GatherDimensionNumbers(
        offset_dims=(), collapsed_slice_dims=(0,), start_index_map=(0,))
    return lax.gather(vec, idx, dnums, (1,),
                      mode=lax.GatherScatterMode.PROMISE_IN_BOUNDS)


def _tc_prep1(x, W1, Ms, Md, Rexp, npad_edges):
    n = x.shape[0]

    def body(x_ref, w_ref, ms_ref, md_ref, rexp_ref, stab_ref, dtab_ref,
             corr_ref):
        h = jnp.dot(x_ref[...], w_ref[...], preferred_element_type=F32)
        a_s = jnp.dot(h, ms_ref[...], preferred_element_type=F32)
        a_d = jnp.dot(h, md_ref[...], preferred_element_type=F32)
        smax = jnp.max(a_s, axis=0, keepdims=True)
        c = _lrelu(a_d + smax)
        z8 = jnp.zeros((n, 8), F32)
        stab_ref[...] = jnp.concatenate([h, a_s, z8], axis=1)
        dtab_ref[...] = jnp.concatenate(
            [a_d, z8, c, jnp.full((n, 8), BIG, F32)], axis=1)
        pex0 = float(npad_edges) * jnp.exp(
            _lrelu(a_s[0:1, :] + a_d[0:1, :]) - c[0:1, :])
        pex0_exp = jnp.dot(pex0, rexp_ref[...],
                           preferred_element_type=F32)
        corr_ref[...] = jnp.concatenate(
            [pex0_exp * h[0:1, :], pex0, jnp.zeros((1, 8), F32)], axis=1)

    return pl.pallas_call(
        body,
        out_shape=(jax.ShapeDtypeStruct((n, 144), F32),
                   jax.ShapeDtypeStruct((n, 32), F32),
                   jax.ShapeDtypeStruct((1, 144), F32)))(x, W1, Ms, Md, Rexp)


def _tc_prep2(accden, corr1, b1, W2, As2t, Ad2t, R16, n, npad_edges):
    def body(ad_ref, corr_ref, b1_ref, w2_ref, as2_ref, ad2_ref, r16_ref,
             stab_ref, dtab_ref, corr2_ref):
        A = ad_ref[0] + ad_ref[1]
        A = A - jnp.concatenate(
            [corr_ref[...], jnp.zeros((n - 1, 144), F32)], axis=0)
        acc = A[:, 0:128]
        den = A[:, 128:144]
        den_exp = jnp.dot(den, r16_ref[...], preferred_element_type=F32)
        h1 = acc / (den_exp + 1e-16) + b1_ref[...]
        h1 = jnp.where(h1 > 0, h1, jnp.exp(h1) - 1.0)
        h2 = jnp.dot(h1, w2_ref[...], preferred_element_type=F32)
        a2s = jnp.dot(h2, as2_ref[...], preferred_element_type=F32)
        a2d = jnp.dot(h2, ad2_ref[...], preferred_element_type=F32)
        smax = jnp.max(a2s)
        c2 = _lrelu(a2d + smax)
        stab_ref[...] = jnp.concatenate(
            [h2, a2s, jnp.zeros((n, 7), F32)], axis=1)
        dtab_ref[...] = jnp.concatenate(
            [jnp.zeros((n, 8), F32), a2d, jnp.zeros((n, 7), F32),
             jnp.full((n, 8), BIG, F32), c2, jnp.full((n, 7), BIG, F32)],
            axis=1)
        pe0 = float(npad_edges) * jnp.exp(
            _lrelu(a2s[0:1, :] + a2d[0:1, :]) - c2[0:1, :])
        corr2_ref[...] = jnp.concatenate(
            [pe0 * h2[0:1, :], jnp.broadcast_to(pe0, (1, 8))], axis=1)

    return pl.pallas_call(
        body,
        out_shape=(jax.ShapeDtypeStruct((n, 48), F32),
                   jax.ShapeDtypeStruct((n, 32), F32),
                   jax.ShapeDtypeStruct((1, 48), F32)))(
            accden, corr1, b1, W2, As2t, Ad2t, R16)


def _tc_final(accden, corr2, b2, Rden, n):
    def body(ad_ref, corr_ref, b2_ref, rd_ref, out_ref):
        A = ad_ref[0] + ad_ref[1]
        A = A - jnp.concatenate(
            [corr_ref[...], jnp.zeros((n - 1, 48), F32)], axis=0)
        acc = A[:, 0:40]
        den = jnp.dot(A[:, 32:48], rd_ref[...], preferred_element_type=F32)
        out_ref[...] = acc / (den + 1e-16) + b2_ref[...]

    return pl.pallas_call(
        body, out_shape=jax.ShapeDtypeStruct((n, 40), F32))(
            accden, corr2, b2, Rden)


def _sc_edge1(stab, dtab, pk, n_pad, chunks):
    rows_pt = n_pad // NS
    mesh = plsc.VectorSubcoreMesh(core_axis_name="c", subcore_axis_name="s")

    @functools.partial(
        pl.kernel,
        out_type=jax.ShapeDtypeStruct((NC, n_pad, 144), F32),
        mesh=mesh,
        compiler_params=pltpu.CompilerParams(use_tc_tiling_on_sc=False),
        scratch_types=[
            pltpu.VMEM((chunks, K), I32),
            pltpu.VMEM((K,), I32),
            pltpu.VMEM((K,), I32),
            pltpu.VMEM((K,), I32),
            pltpu.VMEM((K,), I32),
            pltpu.VMEM((K, 144), F32),
            pltpu.VMEM((K, 144), F32),
            pltpu.VMEM((K, 32), F32),
            pltpu.VMEM((K, 32), F32),
            pltpu.VMEM_SHARED((n_pad, 144), F32),
            pltpu.SemaphoreType.DMA,
            pltpu.SemaphoreType.DMA,
            pltpu.SemaphoreType.DMA,
            pltpu.SemaphoreType.DMA,
        ])
    def kfn(stab_hbm, dtab_hbm, pk_hbm, out_hbm,
            pkbig, sidx0, sidx1, didx0, didx1, srows0, srows1,
            drows0, drows1, acc, ss0, ss1, sd0, sd1):
        cid = lax.axis_index("c")
        sid = lax.axis_index("s")
        wid = cid * NS + sid
        zero = jnp.zeros((16,), F32)
        sidx = (sidx0, sidx1)
        didx = (didx0, didx1)
        srows = (srows0, srows1)
        drows = (drows0, drows1)
        ssem = (ss0, ss1)
        dsem = (sd0, sd1)

        @plsc.parallel_loop(0, K, unroll=8)
        def zbody(r):
            for cc in range(9):
                srows0[r, pl.ds(cc * 16, 16)] = zero
        base0 = sid * rows_pt
        nfull, rem = rows_pt // K, rows_pt % K
        for i in range(nfull):
            pltpu.sync_copy(srows0, acc.at[pl.ds(base0 + i * K, K)])
        if rem:
            pltpu.sync_copy(srows0.at[pl.ds(0, rem)],
                            acc.at[pl.ds(base0 + nfull * K, rem)])
        plsc.subcore_barrier()

        pltpu.sync_copy(pk_hbm.at[wid], pkbig)

        def unpack_issue(g, b):
            for l in range(K // 16):
                v = pkbig[g, pl.ds(l * 16, 16)]
                sidx[b][pl.ds(l * 16, 16)] = lax.shift_right_logical(v, 14)
                didx[b][pl.ds(l * 16, 16)] = lax.bitwise_and(v, 16383)
            pltpu.async_copy(stab_hbm.at[sidx[b]], srows[b], ssem[b])
            pltpu.async_copy(dtab_hbm.at[didx[b]], drows[b], dsem[b])

        def compute_scatter(b):
            pltpu.make_async_copy(stab_hbm.at[sidx[b]], srows[b],
                                  ssem[b]).wait()
            pltpu.make_async_copy(dtab_hbm.at[didx[b]], drows[b],
                                  dsem[b]).wait()

            @plsc.parallel_loop(0, K, unroll=4)
            def ebody(j):
                av = srows[b][j, pl.ds(128, 16)]
                d0 = drows[b][j, pl.ds(0, 16)]
                d1 = drows[b][j, pl.ds(16, 16)]
                s = av + d0
                ex = jnp.exp(jnp.where(s > 0, s, 0.2 * s) - d1)
                srows[b][j, pl.ds(128, 16)] = ex
                for hh in range(8):
                    exb = _bcast_lane(ex, hh)
                    srows[b][j, pl.ds(hh * 16, 16)] = (
                        srows[b][j, pl.ds(hh * 16, 16)] * exb)
            pltpu.sync_copy(srows[b], acc.at[didx[b]], add=True)

        unpack_issue(0, 0)

        def pair(i, carry):
            unpack_issue(2 * i + 1, 1)
            compute_scatter(0)
            unpack_issue(lax.rem(2 * i + 2, chunks), 0)
            compute_scatter(1)
            return carry
        lax.fori_loop(0, chunks // 2, pair, 0)

        pltpu.make_async_copy(stab_hbm.at[sidx0], srows0, ss0).wait()
        pltpu.make_async_copy(dtab_hbm.at[didx0], drows0, sd0).wait()

        plsc.subcore_barrier()
        pltpu.sync_copy(acc.at[pl.ds(base0, rows_pt)],
                        out_hbm.at[cid, pl.ds(base0, rows_pt)])

    return kfn(stab, dtab, pk)


def _sc_edge2(stab, dtab, pk, n_pad, chunks):
    rows_pt = n_pad // NS
    mesh = plsc.VectorSubcoreMesh(core_axis_name="c", subcore_axis_name="s")

    @functools.partial(
        pl.kernel,
        out_type=jax.ShapeDtypeStruct((NC, n_pad, 48), F32),
        mesh=mesh,
        compiler_params=pltpu.CompilerParams(use_tc_tiling_on_sc=False),
        scratch_types=[
            pltpu.VMEM((chunks, K), I32),
            pltpu.VMEM((K,), I32),
            pltpu.VMEM((K,), I32),
            pltpu.VMEM((K,), I32),
            pltpu.VMEM((K,), I32),
            pltpu.VMEM((K, 48), F32),
            pltpu.VMEM((K, 48), F32),
            pltpu.VMEM((K, 32), F32),
            pltpu.VMEM((K, 32), F32),
            pltpu.VMEM_SHARED((n_pad, 48), F32),
            pltpu.SemaphoreType.DMA,
            pltpu.SemaphoreType.DMA,
            pltpu.SemaphoreType.DMA,
            pltpu.SemaphoreType.DMA,
        ])
    def kfn(stab_hbm, dtab_hbm, pk_hbm, out_hbm,
            pkbig, sidx0, sidx1, didx0, didx1, srows0, srows1,
            drows0, drows1, acc, ss0, ss1, sd0, sd1):
        cid = lax.axis_index("c")
        sid = lax.axis_index("s")
        wid = cid * NS + sid
        zero = jnp.zeros((16,), F32)
        sidx = (sidx0, sidx1)
        didx = (didx0, didx1)
        srows = (srows0, srows1)
        drows = (drows0, drows1)
        ssem = (ss0, ss1)
        dsem = (sd0, sd1)

        @plsc.parallel_loop(0, K, unroll=8)
        def zbody(r):
            for cc in range(3):
                srows0[r, pl.ds(cc * 16, 16)] = zero
        base0 = sid * rows_pt
        nfull, rem = rows_pt // K, rows_pt % K
        for i in range(nfull):
            pltpu.sync_copy(srows0, acc.at[pl.ds(base0 + i * K, K)])
        if rem:
            pltpu.sync_copy(srows0.at[pl.ds(0, rem)],
                            acc.at[pl.ds(base0 + nfull * K, rem)])
        plsc.subcore_barrier()

        pltpu.sync_copy(pk_hbm.at[wid], pkbig)

        def unpack_issue(g, b):
            for l in range(K // 16):
                v = pkbig[g, pl.ds(l * 16, 16)]
                sidx[b][pl.ds(l * 16, 16)] = lax.shift_right_logical(v, 14)
                didx[b][pl.ds(l * 16, 16)] = lax.bitwise_and(v, 16383)
            pltpu.async_copy(stab_hbm.at[sidx[b]], srows[b], ssem[b])
            pltpu.async_copy(dtab_hbm.at[didx[b]], drows[b], dsem[b])

        def compute_scatter(b):
            pltpu.make_async_copy(stab_hbm.at[sidx[b]], srows[b],
                                  ssem[b]).wait()
            pltpu.make_async_copy(dtab_hbm.at[didx[b]], drows[b],
                                  dsem[b]).wait()
            lanes = lax.iota(I32, 16)

            @plsc.parallel_loop(0, K, unroll=4)
            def ebody(j):
                sv2 = srows[b][j, pl.ds(32, 16)]
                d0 = drows[b][j, pl.ds(0, 16)]
                d1 = drows[b][j, pl.ds(16, 16)]
                t = sv2 + d0
                e0 = jnp.exp(jnp.where(t > 0, t, 0.2 * t) - d1)
                exv = _bcast_lane(e0, 8)
                srows[b][j, pl.ds(0, 16)] = srows[b][j, pl.ds(0, 16)] * exv
                srows[b][j, pl.ds(16, 16)] = srows[b][j, pl.ds(16, 16)] * exv
                srows[b][j, pl.ds(32, 16)] = jnp.where(
                    lanes < 8, sv2 * exv, exv)
            pltpu.sync_copy(srows[b], acc.at[didx[b]], add=True)

        unpack_issue(0, 0)

        def pair(i, carry):
            unpack_issue(2 * i + 1, 1)
            compute_scatter(0)
            unpack_issue(lax.rem(2 * i + 2, chunks), 0)
            compute_scatter(1)
            return carry
        lax.fori_loop(0, chunks // 2, pair, 0)

        pltpu.make_async_copy(stab_hbm.at[sidx0], srows0, ss0).wait()
        pltpu.make_async_copy(dtab_hbm.at[didx0], drows0, sd0).wait()

        plsc.subcore_barrier()
        pltpu.sync_copy(acc.at[pl.ds(base0, rows_pt)],
                        out_hbm.at[cid, pl.ds(base0, rows_pt)])

    return kfn(stab, dtab, pk)


def kernel(x, edge_index, W1, att_src1, att_dst1, b1,
           W2, att_src2, att_dst2, b2):
    n = x.shape[0]
    e = edge_index.shape[1]
    heads, hid = att_src1.shape

    et = e + n
    chunks = (et + NW * K - 1) // (NW * K)
    chunks += chunks % 2
    ep = chunks * NW * K
    npad_edges = ep - et

    loops = jnp.arange(n, dtype=I32)
    padi = jnp.zeros((npad_edges,), I32)
    src = jnp.concatenate([edge_index[0].astype(I32), loops, padi])
    dst = jnp.concatenate([edge_index[1].astype(I32), loops, padi])
    pk = jnp.bitwise_or(jnp.left_shift(src, 14), dst).reshape(NW, chunks, K)

    eye8 = jnp.eye(heads, dtype=F32)
    Ms1 = (att_src1.astype(F32)[:, :, None] * eye8[:, None, :]).reshape(
        heads * hid, heads)
    Md1 = (att_dst1.astype(F32)[:, :, None] * eye8[:, None, :]).reshape(
        heads * hid, heads)
    Rexp = jnp.repeat(eye8, hid, axis=1)
    R16 = jnp.concatenate(
        [Rexp, jnp.zeros((8, heads * hid), F32)], axis=0)
    Rden = jnp.concatenate(
        [jnp.zeros((8, 40), F32), jnp.full((8, 40), 1.0 / 8.0, F32)], axis=0)

    stab1, dtab1, corr1 = _tc_prep1(x, W1, Ms1, Md1, Rexp, npad_edges)
    accden1 = _sc_edge1(stab1, dtab1, pk, n, chunks)
    stab2, dtab2, corr2 = _tc_prep2(accden1, corr1, b1, W2,
                                    att_src2.T.astype(F32),
                                    att_dst2.T.astype(F32), R16, n,
                                    npad_edges)
    accden2 = _sc_edge2(stab2, dtab2, pk, n, chunks)
    return _tc_final(accden2, corr2, b2, Rden, n)
